# baseline (speedup 1.0000x reference)
import jax
import jax.numpy as jnp
from jax import lax
from jax.experimental import pallas as pl
from jax.experimental.pallas import tpu as pltpu
from jax.experimental.pallas import tpu_sc as plsc

N_WORDS = 1000000
D = 64
B = 16384
NC = 2
NS = 16
NW = NC * NS
PER_W = B // NW
CHUNK = 128
NCHUNK = PER_W // CHUNK

BW = 8192
NBLK = 123
WROWS = NBLK * BW // 2


def _fmt_body(wt_ref, w2_ref):
    x = wt_ref[...]
    lo = lax.slice(x, (0, 0), (D, BW // 2))
    hi = lax.slice(x, (0, BW // 2), (D, BW))
    w2_ref[...] = jnp.concatenate(
        [jnp.transpose(lo, (1, 0)), jnp.transpose(hi, (1, 0))], axis=1)


def _format_word_table(word_emb_table):
    wtT = word_emb_table.T
    return pl.pallas_call(
        _fmt_body,
        grid=(NBLK,),
        in_specs=[
            pl.BlockSpec((D, BW), lambda i: (0, i)),
        ],
        out_specs=pl.BlockSpec((BW // 2, 2 * D), lambda i: (i, 0)),
        out_shape=jax.ShapeDtypeStruct((WROWS, 2 * D), jnp.float32),
    )(wtT)


TPW = 248
TCHUNK = 8
NTCH = TPW // TCHUNK
WPC = TCHUNK * 128
NFULL = 7812
PADN = 1000448


def _proj_body(fc_hbm, btT_hbm, p_hbm, fc_v, slab, tslab, acc, sem):
    wid = lax.axis_index("s") * NC + lax.axis_index("c")
    start_tile = jnp.minimum(wid * 244, NFULL - TPW)
    base = start_tile * 128
    pltpu.sync_copy(fc_hbm, fc_v)
    fc_seg = [fc_v[pl.ds(16 * k, 16)] for k in range(4)]
    fcs = [fc_seg[i // 16][i % 16] for i in range(D)]

    def chunk(ch, _):
        c0 = pl.multiple_of(base + ch * WPC, 8)
        for r in range(8):
            pltpu.async_copy(
                btT_hbm.at[pl.ds(8 * r, 8), pl.ds(c0, WPC)], slab.at[r], sem
            ).wait()

            def grp(g, _, r=r):
                seg = pl.ds(pl.multiple_of(g * 16, 16), 16)
                s = acc[seg] if r else jnp.zeros((16,), jnp.float32)
                for d in range(8):
                    s = s + slab[r, d, seg] * fcs[8 * r + d]
                acc[seg] = s
                return 0

            lax.fori_loop(0, WPC // 16, grp, 0)
        pltpu.sync_copy(acc, p_hbm.at[pl.ds(c0, WPC)])
        return 0

    lax.fori_loop(0, NTCH, chunk, 0)

    @pl.when(wid == NW - 1)
    def _tail():
        t0 = NFULL * 128
        for r in range(8):
            pltpu.async_copy(
                btT_hbm.at[pl.ds(8 * r, 8), pl.ds(t0, 64)], tslab.at[r], sem
            ).wait()
            for g in range(4):
                seg = pl.ds(g * 16, 16)
                s = acc[seg] if r else jnp.zeros((16,), jnp.float32)
                for d in range(8):
                    s = s + tslab[r, d, seg] * fcs[8 * r + d]
                acc[seg] = s
        pltpu.sync_copy(acc.at[pl.ds(0, 64)], p_hbm.at[pl.ds(t0, 64)])


def _bias_projection(bias_table, fc_w):
    btT = bias_table.T
    mesh = plsc.VectorSubcoreMesh(core_axis_name="c", subcore_axis_name="s",
                                  num_cores=NC, num_subcores=NS)
    run = pl.kernel(
        _proj_body,
        out_type=jax.ShapeDtypeStruct((PADN,), jnp.float32),
        mesh=mesh,
        compiler_params=pltpu.CompilerParams(needs_layout_passes=False,
                                             use_tc_tiling_on_sc=True),
        scratch_types=[
            pltpu.VMEM((D,), jnp.float32),
            pltpu.VMEM((8, 8, WPC), jnp.float32),
            pltpu.VMEM((8, 8, 64), jnp.float32),
            pltpu.VMEM((WPC,), jnp.float32),
            pltpu.SemaphoreType.DMA,
        ],
    )
    return run(fc_w.reshape(D), btT)


def _sc_body(wid_hbm, cid_hbm, wt2_hbm, p_hbm, fc_hbm, out_hbm,
             idx_w, idx_c, idxh_w, idxh_c, ww, wc, pw, pc, fc_v, out_v, sem):
    wid = lax.axis_index("s") * NC + lax.axis_index("c")
    base = wid * PER_W

    for j in range(NCHUNK):
        pltpu.sync_copy(wid_hbm.at[pl.ds(base + j * CHUNK, CHUNK)], idx_w.at[j])
        pltpu.sync_copy(cid_hbm.at[pl.ds(base + j * CHUNK, CHUNK)], idx_c.at[j])
    pltpu.sync_copy(fc_hbm, fc_v)

    for j in range(NCHUNK):
        for g in range(CHUNK // 16):
            seg = pl.ds(g * 16, 16)
            vw = idx_w[j, seg]
            vc = idx_c[j, seg]
            idxh_w[j, seg] = lax.shift_left(lax.shift_right_logical(vw, 13), 12) | (vw & 4095)
            idxh_c[j, seg] = lax.shift_left(lax.shift_right_logical(vc, 13), 12) | (vc & 4095)

    fc_seg = [fc_v[pl.ds(16 * k, 16)] for k in range(4)]
    lane = lax.iota(jnp.int32, 16)
    hone = jnp.full((16,), 1, jnp.int32)

    for c in range(NCHUNK):
        cps = [
            pltpu.async_copy(wt2_hbm.at[idxh_w.at[c]], ww, sem),
            pltpu.async_copy(wt2_hbm.at[idxh_c.at[c]], wc, sem),
            pltpu.async_copy(p_hbm.at[idx_w.at[c]], pw, sem),
            pltpu.async_copy(p_hbm.at[idx_c.at[c]], pc, sem),
        ]
        for cp in cps:
            cp.wait()

        def grp(g, _, c=c):
            seg = pl.ds(pl.multiple_of(g * 16, 16), 16)
            offs_w = (lax.shift_right_logical(idx_w[c, seg], 12) & hone) * 64
            offs_c = (lax.shift_right_logical(idx_c[c, seg], 12) & hone) * 64
            acc = jnp.zeros((16,), jnp.float32)
            for l in range(16):
                b = g * 16 + l
                ow = pl.multiple_of(offs_w[l], 8)
                oc = pl.multiple_of(offs_c[l], 8)
                s = jnp.zeros((16,), jnp.float32)
                for k in range(4):
                    s = s + (ww[b, pl.ds(ow + 16 * k, 16)]
                             * wc[b, pl.ds(oc + 16 * k, 16)]) * fc_seg[k]
                acc = jnp.where(lane == l, jnp.sum(s), acc)
            acc = acc + pw[seg] + pc[seg]
            acc = 1.0 / (1.0 + jnp.exp(-acc))
            out_v[pl.ds(pl.multiple_of(c * CHUNK + g * 16, 16), 16)] = acc
            return 0

        lax.fori_loop(0, CHUNK // 16, grp, 0)

    pltpu.sync_copy(out_v, out_hbm.at[pl.ds(base, PER_W)])


@jax.jit
def kernel(word_ids, context_ids, word_emb_table, bias_table, fc_w):
    wt2 = _format_word_table(word_emb_table)
    p = _bias_projection(bias_table, fc_w)
    mesh = plsc.VectorSubcoreMesh(core_axis_name="c", subcore_axis_name="s",
                                  num_cores=NC, num_subcores=NS)
    run = pl.kernel(
        _sc_body,
        out_type=jax.ShapeDtypeStruct((B,), jnp.float32),
        mesh=mesh,
        compiler_params=pltpu.CompilerParams(needs_layout_passes=False,
                                             use_tc_tiling_on_sc=False),
        scratch_types=[
            pltpu.VMEM((NCHUNK, CHUNK), jnp.int32),
            pltpu.VMEM((NCHUNK, CHUNK), jnp.int32),
            pltpu.VMEM((NCHUNK, CHUNK), jnp.int32),
            pltpu.VMEM((NCHUNK, CHUNK), jnp.int32),
            pltpu.VMEM((CHUNK, 2 * D), jnp.float32),
            pltpu.VMEM((CHUNK, 2 * D), jnp.float32),
            pltpu.VMEM((CHUNK,), jnp.float32),
            pltpu.VMEM((CHUNK,), jnp.float32),
            pltpu.VMEM((D,), jnp.float32),
            pltpu.VMEM((PER_W,), jnp.float32),
            pltpu.SemaphoreType.DMA,
        ],
    )
    out = run(word_ids.astype(jnp.int32), context_ids.astype(jnp.int32),
              wt2, p, fc_w.reshape(D))
    return out.reshape(B, 1)


# speedup vs baseline: 1.7245x; 1.7245x over previous
import jax
import jax.numpy as jnp
from jax import lax
from jax.experimental import pallas as pl
from jax.experimental.pallas import tpu as pltpu
from jax.experimental.pallas import tpu_sc as plsc

N_WORDS = 1000000
D = 64
B = 16384
NC = 2
NS = 16
NW = NC * NS
PER_W = B // NW
CHUNK = 128
NCHUNK = PER_W // CHUNK

BW = 16384
NBLK = 62
PN = NBLK * BW
WROWS = NBLK * BW // 2


def _fmt_body(fc_ref, wt_ref, bt_ref, w2_ref, p_ref):
    x = wt_ref[...]
    lo = lax.slice(x, (0, 0), (D, BW // 2))
    hi = lax.slice(x, (0, BW // 2), (D, BW))
    w2_ref[...] = jnp.concatenate(
        [jnp.transpose(lo, (1, 0)), jnp.transpose(hi, (1, 0))], axis=1)
    p_ref[...] = jnp.sum(bt_ref[...] * fc_ref[...], axis=0)


def _format_and_project(word_emb_table, bias_table, fc_w):
    wtT = word_emb_table.T
    btT = bias_table.T
    fc_col = fc_w.reshape(D, 1)
    return pl.pallas_call(
        _fmt_body,
        grid=(NBLK,),
        in_specs=[
            pl.BlockSpec((D, 1), lambda i: (0, 0)),
            pl.BlockSpec((D, BW), lambda i: (0, i)),
            pl.BlockSpec((D, BW), lambda i: (0, i)),
        ],
        out_specs=[
            pl.BlockSpec((BW // 2, 2 * D), lambda i: (i, 0)),
            pl.BlockSpec((BW,), lambda i: (i,)),
        ],
        out_shape=[
            jax.ShapeDtypeStruct((WROWS, 2 * D), jnp.float32),
            jax.ShapeDtypeStruct((PN,), jnp.float32),
        ],
    )(fc_col, wtT, btT)


def _sc_body(wid_hbm, cid_hbm, wt2_hbm, p_hbm, fc_hbm, out_hbm,
             idx_w, idx_c, idxh_w, idxh_c, ww, wc, pw, pc, fc_v, out_v, sem):
    wid = lax.axis_index("s") * NC + lax.axis_index("c")
    base = wid * PER_W

    for j in range(NCHUNK):
        pltpu.sync_copy(wid_hbm.at[pl.ds(base + j * CHUNK, CHUNK)], idx_w.at[j])
        pltpu.sync_copy(cid_hbm.at[pl.ds(base + j * CHUNK, CHUNK)], idx_c.at[j])
    pltpu.sync_copy(fc_hbm, fc_v)

    for j in range(NCHUNK):
        for g in range(CHUNK // 16):
            seg = pl.ds(g * 16, 16)
            vw = idx_w[j, seg]
            vc = idx_c[j, seg]
            idxh_w[j, seg] = lax.shift_left(lax.shift_right_logical(vw, 14), 13) | (vw & 8191)
            idxh_c[j, seg] = lax.shift_left(lax.shift_right_logical(vc, 14), 13) | (vc & 8191)

    fc_seg = [fc_v[pl.ds(16 * k, 16)] for k in range(4)]
    lane = lax.iota(jnp.int32, 16)
    hone = jnp.full((16,), 1, jnp.int32)

    for c in range(NCHUNK):
        cps = [
            pltpu.async_copy(wt2_hbm.at[idxh_w.at[c]], ww, sem),
            pltpu.async_copy(wt2_hbm.at[idxh_c.at[c]], wc, sem),
            pltpu.async_copy(p_hbm.at[idx_w.at[c]], pw, sem),
            pltpu.async_copy(p_hbm.at[idx_c.at[c]], pc, sem),
        ]
        for cp in cps:
            cp.wait()

        def grp(g, _, c=c):
            seg = pl.ds(pl.multiple_of(g * 16, 16), 16)
            offs_w = (lax.shift_right_logical(idx_w[c, seg], 13) & hone) * 64
            offs_c = (lax.shift_right_logical(idx_c[c, seg], 13) & hone) * 64
            acc = jnp.zeros((16,), jnp.float32)
            for l in range(16):
                b = g * 16 + l
                ow = pl.multiple_of(offs_w[l], 8)
                oc = pl.multiple_of(offs_c[l], 8)
                s = jnp.zeros((16,), jnp.float32)
                for k in range(4):
                    s = s + (ww[b, pl.ds(ow + 16 * k, 16)]
                             * wc[b, pl.ds(oc + 16 * k, 16)]) * fc_seg[k]
                acc = jnp.where(lane == l, jnp.sum(s), acc)
            acc = acc + pw[seg] + pc[seg]
            acc = 1.0 / (1.0 + jnp.exp(-acc))
            out_v[pl.ds(pl.multiple_of(c * CHUNK + g * 16, 16), 16)] = acc
            return 0

        lax.fori_loop(0, CHUNK // 16, grp, 0)

    pltpu.sync_copy(out_v, out_hbm.at[pl.ds(base, PER_W)])


@jax.jit
def kernel(word_ids, context_ids, word_emb_table, bias_table, fc_w):
    wt2, p = _format_and_project(word_emb_table, bias_table, fc_w)
    mesh = plsc.VectorSubcoreMesh(core_axis_name="c", subcore_axis_name="s",
                                  num_cores=NC, num_subcores=NS)
    run = pl.kernel(
        _sc_body,
        out_type=jax.ShapeDtypeStruct((B,), jnp.float32),
        mesh=mesh,
        compiler_params=pltpu.CompilerParams(needs_layout_passes=False,
                                             use_tc_tiling_on_sc=False),
        scratch_types=[
            pltpu.VMEM((NCHUNK, CHUNK), jnp.int32),
            pltpu.VMEM((NCHUNK, CHUNK), jnp.int32),
            pltpu.VMEM((NCHUNK, CHUNK), jnp.int32),
            pltpu.VMEM((NCHUNK, CHUNK), jnp.int32),
            pltpu.VMEM((CHUNK, 2 * D), jnp.float32),
            pltpu.VMEM((CHUNK, 2 * D), jnp.float32),
            pltpu.VMEM((CHUNK,), jnp.float32),
            pltpu.VMEM((CHUNK,), jnp.float32),
            pltpu.VMEM((D,), jnp.float32),
            pltpu.VMEM((PER_W,), jnp.float32),
            pltpu.SemaphoreType.DMA,
        ],
    )
    out = run(word_ids.astype(jnp.int32), context_ids.astype(jnp.int32),
              wt2, p, fc_w.reshape(D))
    return out.reshape(B, 1)


# speedup vs baseline: 1.8694x; 1.0840x over previous
import jax
import jax.numpy as jnp
from jax import lax
from jax.experimental import pallas as pl
from jax.experimental.pallas import tpu as pltpu
from jax.experimental.pallas import tpu_sc as plsc

N_WORDS = 1000000
D = 64
B = 16384
NC = 2
NS = 16
NW = NC * NS
PER_W = B // NW
CHUNK = 128
NCHUNK = PER_W // CHUNK

BW = 32768
NBLK = 31
PN = NBLK * BW
WROWS = NBLK * BW // 2


def _fmt_body(fc_ref, wt_ref, bt_ref, w2_ref, p_ref):
    x = wt_ref[...]
    lo = lax.slice(x, (0, 0), (D, BW // 2))
    hi = lax.slice(x, (0, BW // 2), (D, BW))
    w2_ref[...] = jnp.concatenate(
        [jnp.transpose(lo, (1, 0)), jnp.transpose(hi, (1, 0))], axis=1)
    p_ref[...] = jnp.sum(bt_ref[...] * fc_ref[...], axis=0)


def _format_and_project(word_emb_table, bias_table, fc_w):
    wtT = word_emb_table.T
    btT = bias_table.T
    fc_col = fc_w.reshape(D, 1)
    return pl.pallas_call(
        _fmt_body,
        grid=(NBLK,),
        in_specs=[
            pl.BlockSpec((D, 1), lambda i: (0, 0)),
            pl.BlockSpec((D, BW), lambda i: (0, i)),
            pl.BlockSpec((D, BW), lambda i: (0, i)),
        ],
        out_specs=[
            pl.BlockSpec((BW // 2, 2 * D), lambda i: (i, 0)),
            pl.BlockSpec((BW,), lambda i: (i,)),
        ],
        out_shape=[
            jax.ShapeDtypeStruct((WROWS, 2 * D), jnp.float32),
            jax.ShapeDtypeStruct((PN,), jnp.float32),
        ],
        compiler_params=pltpu.CompilerParams(vmem_limit_bytes=100 * 1024 * 1024),
    )(fc_col, wtT, btT)


def _sc_body(wid_hbm, cid_hbm, wt2_hbm, p_hbm, fc_hbm, out_hbm,
             idx_w, idx_c, idxh_w, idxh_c, ww, wc, pw, pc, fc_v, out_v, sem):
    wid = lax.axis_index("s") * NC + lax.axis_index("c")
    base = wid * PER_W

    for j in range(NCHUNK):
        pltpu.sync_copy(wid_hbm.at[pl.ds(base + j * CHUNK, CHUNK)], idx_w.at[j])
        pltpu.sync_copy(cid_hbm.at[pl.ds(base + j * CHUNK, CHUNK)], idx_c.at[j])
    pltpu.sync_copy(fc_hbm, fc_v)

    for j in range(NCHUNK):
        for g in range(CHUNK // 16):
            seg = pl.ds(g * 16, 16)
            vw = idx_w[j, seg]
            vc = idx_c[j, seg]
            idxh_w[j, seg] = lax.shift_left(lax.shift_right_logical(vw, 15), 14) | (vw & 16383)
            idxh_c[j, seg] = lax.shift_left(lax.shift_right_logical(vc, 15), 14) | (vc & 16383)

    fc_seg = [fc_v[pl.ds(16 * k, 16)] for k in range(4)]
    lane = lax.iota(jnp.int32, 16)
    hone = jnp.full((16,), 1, jnp.int32)

    for c in range(NCHUNK):
        cps = [
            pltpu.async_copy(wt2_hbm.at[idxh_w.at[c]], ww, sem),
            pltpu.async_copy(wt2_hbm.at[idxh_c.at[c]], wc, sem),
            pltpu.async_copy(p_hbm.at[idx_w.at[c]], pw, sem),
            pltpu.async_copy(p_hbm.at[idx_c.at[c]], pc, sem),
        ]
        for cp in cps:
            cp.wait()

        def grp(g, _, c=c):
            seg = pl.ds(pl.multiple_of(g * 16, 16), 16)
            offs_w = (lax.shift_right_logical(idx_w[c, seg], 14) & hone) * 64
            offs_c = (lax.shift_right_logical(idx_c[c, seg], 14) & hone) * 64
            acc = jnp.zeros((16,), jnp.float32)
            for l in range(16):
                b = g * 16 + l
                ow = pl.multiple_of(offs_w[l], 8)
                oc = pl.multiple_of(offs_c[l], 8)
                s = jnp.zeros((16,), jnp.float32)
                for k in range(4):
                    s = s + (ww[b, pl.ds(ow + 16 * k, 16)]
                             * wc[b, pl.ds(oc + 16 * k, 16)]) * fc_seg[k]
                acc = jnp.where(lane == l, jnp.sum(s), acc)
            acc = acc + pw[seg] + pc[seg]
            acc = 1.0 / (1.0 + jnp.exp(-acc))
            out_v[pl.ds(pl.multiple_of(c * CHUNK + g * 16, 16), 16)] = acc
            return 0

        lax.fori_loop(0, CHUNK // 16, grp, 0)

    pltpu.sync_copy(out_v, out_hbm.at[pl.ds(base, PER_W)])


@jax.jit
def kernel(word_ids, context_ids, word_emb_table, bias_table, fc_w):
    wt2, p = _format_and_project(word_emb_table, bias_table, fc_w)
    mesh = plsc.VectorSubcoreMesh(core_axis_name="c", subcore_axis_name="s",
                                  num_cores=NC, num_subcores=NS)
    run = pl.kernel(
        _sc_body,
        out_type=jax.ShapeDtypeStruct((B,), jnp.float32),
        mesh=mesh,
        compiler_params=pltpu.CompilerParams(needs_layout_passes=False,
                                             use_tc_tiling_on_sc=False),
        scratch_types=[
            pltpu.VMEM((NCHUNK, CHUNK), jnp.int32),
            pltpu.VMEM((NCHUNK, CHUNK), jnp.int32),
            pltpu.VMEM((NCHUNK, CHUNK), jnp.int32),
            pltpu.VMEM((NCHUNK, CHUNK), jnp.int32),
            pltpu.VMEM((CHUNK, 2 * D), jnp.float32),
            pltpu.VMEM((CHUNK, 2 * D), jnp.float32),
            pltpu.VMEM((CHUNK,), jnp.float32),
            pltpu.VMEM((CHUNK,), jnp.float32),
            pltpu.VMEM((D,), jnp.float32),
            pltpu.VMEM((PER_W,), jnp.float32),
            pltpu.SemaphoreType.DMA,
        ],
    )
    out = run(word_ids.astype(jnp.int32), context_ids.astype(jnp.int32),
              wt2, p, fc_w.reshape(D))
    return out.reshape(B, 1)
